# Initial kernel scaffold; baseline (speedup 1.0000x reference)
import jax
import jax.numpy as jnp
from jax.experimental import pallas as pl


def kernel(inputs, table):
    raise NotImplementedError("write your pallas kernel here")


# speedup vs baseline: 2.9618x; 2.9618x over previous
import functools

import jax
import jax.numpy as jnp
from jax import lax
from jax.experimental import pallas as pl
from jax.experimental.pallas import tpu as pltpu
from jax.experimental.pallas import tpu_sc as plsc

VOCAB = 100000
EMBED_DIM = 128
BATCH = 4096
HIST = 50

NUM_CORES = 2
NUM_SUBCORES = 16
NW = NUM_CORES * NUM_SUBCORES
TOTAL = BATCH * HIST
BPW = TOTAL // NW
CHUNK = 128
NCHUNK = BPW // CHUNK

_mesh = plsc.VectorSubcoreMesh(core_axis_name="c", subcore_axis_name="s")


@functools.partial(
    pl.kernel,
    mesh=_mesh,
    out_type=jax.ShapeDtypeStruct((TOTAL, EMBED_DIM), jnp.float32),
    scratch_types=[
        pltpu.VMEM((NCHUNK, CHUNK), jnp.int32),
        pltpu.VMEM((CHUNK, EMBED_DIM), jnp.float32),
        pltpu.SemaphoreType.DMA,
    ],
)
def _embedding_lookup(idx_hbm, table_hbm, out_hbm, idx_v, rows_v, sem):
    wid = lax.axis_index("s") * NUM_CORES + lax.axis_index("c")
    base = wid * BPW
    pltpu.sync_copy(idx_hbm.at[wid], idx_v)

    def body(c, carry):
        pltpu.async_copy(table_hbm.at[idx_v.at[c]], rows_v, sem).wait()
        pltpu.sync_copy(rows_v, out_hbm.at[pl.ds(base + c * CHUNK, CHUNK)])
        return carry

    lax.fori_loop(0, NCHUNK, body, 0)


def kernel(inputs, table):
    idx = inputs.astype(jnp.int32).reshape(NW, NCHUNK, CHUNK)
    out = _embedding_lookup(idx, table)
    return out.reshape(BATCH, HIST, EMBED_DIM)


# speedup vs baseline: 3.2839x; 1.1087x over previous
import functools

import jax
import jax.numpy as jnp
from jax import lax
from jax.experimental import pallas as pl
from jax.experimental.pallas import tpu as pltpu
from jax.experimental.pallas import tpu_sc as plsc

VOCAB = 100000
EMBED_DIM = 128
BATCH = 4096
HIST = 50

NUM_CORES = 2
NUM_SUBCORES = 16
NW = NUM_CORES * NUM_SUBCORES
TOTAL = BATCH * HIST
BPW = TOTAL // NW
CHUNK = 64
NCHUNK = BPW // CHUNK
NBUF = 4
NGRP = NCHUNK // NBUF

_mesh = plsc.VectorSubcoreMesh(core_axis_name="c", subcore_axis_name="s")


@functools.partial(
    pl.kernel,
    mesh=_mesh,
    out_type=jax.ShapeDtypeStruct((TOTAL, EMBED_DIM), jnp.float32),
    scratch_types=[
        pltpu.VMEM((NCHUNK, CHUNK), jnp.int32),
    ]
    + [pltpu.VMEM((CHUNK, EMBED_DIM), jnp.float32) for _ in range(NBUF)]
    + [pltpu.SemaphoreType.DMA for _ in range(2 * NBUF)],
)
def _embedding_lookup(idx_hbm, table_hbm, out_hbm, idx_v, *bufs_and_sems):
    rows = bufs_and_sems[:NBUF]
    gsem = bufs_and_sems[NBUF:2 * NBUF]
    osem = bufs_and_sems[2 * NBUF:]
    wid = lax.axis_index("s") * NUM_CORES + lax.axis_index("c")
    base = wid * BPW
    pltpu.sync_copy(idx_hbm.at[wid], idx_v)

    def gather(c, j):
        return pltpu.make_async_copy(table_hbm.at[idx_v.at[c]], rows[j], gsem[j])

    def out_copy(c, j):
        return pltpu.make_async_copy(
            rows[j], out_hbm.at[pl.ds(base + c * CHUNK, CHUNK)], osem[j])

    for j in range(NBUF):
        gather(j, j).start()
    for j in range(NBUF):
        gather(j, j).wait()
        out_copy(j, j).start()

    def body(t, carry):
        c0 = t * NBUF
        for j in range(NBUF):
            out_copy(c0 - NBUF + j, j).wait()
            gather(c0 + j, j).start()
        for j in range(NBUF):
            gather(c0 + j, j).wait()
            out_copy(c0 + j, j).start()
        return carry

    lax.fori_loop(1, NGRP, body, 0)

    for j in range(NBUF):
        out_copy((NGRP - 1) * NBUF + j, j).wait()


def kernel(inputs, table):
    idx = inputs.astype(jnp.int32).reshape(NW, NCHUNK, CHUNK)
    out = _embedding_lookup(idx, table)
    return out.reshape(BATCH, HIST, EMBED_DIM)


# speedup vs baseline: 3.3368x; 1.0161x over previous
import functools

import jax
import jax.numpy as jnp
from jax import lax
from jax.experimental import pallas as pl
from jax.experimental.pallas import tpu as pltpu
from jax.experimental.pallas import tpu_sc as plsc

VOCAB = 100000
EMBED_DIM = 128
BATCH = 4096
HIST = 50

NUM_CORES = 2
NUM_SUBCORES = 16
NW = NUM_CORES * NUM_SUBCORES
TOTAL = BATCH * HIST
BPW = TOTAL // NW
CHUNK = 128
NCHUNK = BPW // CHUNK
NBUF = 5
NGRP = NCHUNK // NBUF
LAG = 3

_mesh = plsc.VectorSubcoreMesh(core_axis_name="c", subcore_axis_name="s")


@functools.partial(
    pl.kernel,
    mesh=_mesh,
    out_type=jax.ShapeDtypeStruct((TOTAL, EMBED_DIM), jnp.float32),
    scratch_types=[
        pltpu.VMEM((NCHUNK, CHUNK), jnp.int32),
    ]
    + [pltpu.VMEM((CHUNK, EMBED_DIM), jnp.float32) for _ in range(NBUF)]
    + [pltpu.SemaphoreType.DMA for _ in range(2 * NBUF)],
)
def _embedding_lookup(idx_hbm, table_hbm, out_hbm, idx_v, *bufs_and_sems):
    rows = bufs_and_sems[:NBUF]
    gsem = bufs_and_sems[NBUF:2 * NBUF]
    osem = bufs_and_sems[2 * NBUF:]
    wid = lax.axis_index("s") * NUM_CORES + lax.axis_index("c")
    base = wid * BPW
    pltpu.sync_copy(idx_hbm.at[wid], idx_v)

    def gather(c, j):
        return pltpu.make_async_copy(table_hbm.at[idx_v.at[c]], rows[j], gsem[j])

    def out_copy(c, j):
        return pltpu.make_async_copy(
            rows[j], out_hbm.at[pl.ds(base + c * CHUNK, CHUNK)], osem[j])

    for c in range(NBUF):
        gather(c, c).start()
        if c >= LAG:
            cd = c - LAG
            gather(cd, cd % NBUF).wait()
            out_copy(cd, cd % NBUF).start()

    def body(t, carry):
        c0 = t * NBUF
        for j in range(NBUF):
            c = c0 + j
            out_copy(c - NBUF, j).wait()
            gather(c, j).start()
            cd = c - LAG
            jd = (j - LAG) % NBUF
            gather(cd, jd).wait()
            out_copy(cd, jd).start()
        return carry

    lax.fori_loop(1, NGRP, body, 0)

    for k in range(LAG):
        cd = NCHUNK - LAG + k
        jd = cd % NBUF
        gather(cd, jd).wait()
        out_copy(cd, jd).start()
    for j in range(NBUF):
        out_copy(NCHUNK - NBUF + ((j - NCHUNK) % NBUF), j).wait()


def kernel(inputs, table):
    idx = inputs.astype(jnp.int32).reshape(NW, NCHUNK, CHUNK)
    out = _embedding_lookup(idx, table)
    return out.reshape(BATCH, HIST, EMBED_DIM)


# speedup vs baseline: 5.9529x; 1.7840x over previous
import functools

import jax
import jax.numpy as jnp
from jax import lax
from jax.experimental import pallas as pl
from jax.experimental.pallas import tpu as pltpu
from jax.experimental.pallas import tpu_sc as plsc

VOCAB = 100000
EMBED_DIM = 128
BATCH = 4096
HIST = 50

NUM_CORES = 2
NUM_SUBCORES = 16
NW = NUM_CORES * NUM_SUBCORES
BRW = BATCH // NW
NCHUNK = BRW
NBUF = 8
NGRP = NCHUNK // NBUF
LAG = 4

_mesh = plsc.VectorSubcoreMesh(core_axis_name="c", subcore_axis_name="s")


@functools.partial(
    pl.kernel,
    mesh=_mesh,
    out_type=jax.ShapeDtypeStruct((BATCH, HIST, EMBED_DIM), jnp.float32),
    scratch_types=[
        pltpu.VMEM((NCHUNK, HIST), jnp.int32),
    ]
    + [pltpu.VMEM((HIST, EMBED_DIM), jnp.float32) for _ in range(NBUF)]
    + [pltpu.SemaphoreType.DMA for _ in range(2 * NBUF)],
)
def _embedding_lookup(idx_hbm, table_hbm, out_hbm, idx_v, *bufs_and_sems):
    rows = bufs_and_sems[:NBUF]
    gsem = bufs_and_sems[NBUF:2 * NBUF]
    osem = bufs_and_sems[2 * NBUF:]
    wid = lax.axis_index("s") * NUM_CORES + lax.axis_index("c")
    base = wid * BRW
    pltpu.sync_copy(idx_hbm.at[wid], idx_v)

    def gather(c, j):
        return pltpu.make_async_copy(table_hbm.at[idx_v.at[c]], rows[j], gsem[j])

    def out_copy(c, j):
        return pltpu.make_async_copy(rows[j], out_hbm.at[base + c], osem[j])

    for c in range(NBUF):
        gather(c, c).start()
        if c >= LAG:
            cd = c - LAG
            gather(cd, cd % NBUF).wait()
            out_copy(cd, cd % NBUF).start()

    def body(t, carry):
        c0 = t * NBUF
        for j in range(NBUF):
            c = c0 + j
            out_copy(c - NBUF, j).wait()
            gather(c, j).start()
            cd = c - LAG
            jd = (j - LAG) % NBUF
            gather(cd, jd).wait()
            out_copy(cd, jd).start()
        return carry

    lax.fori_loop(1, NGRP, body, 0)

    for k in range(LAG):
        cd = NCHUNK - LAG + k
        jd = cd % NBUF
        gather(cd, jd).wait()
        out_copy(cd, jd).start()
    for j in range(NBUF):
        out_copy(NCHUNK - NBUF + j, j).wait()


def kernel(inputs, table):
    idx = inputs.astype(jnp.int32).reshape(NW, NCHUNK, HIST)
    return _embedding_lookup(idx, table)


# speedup vs baseline: 5.9572x; 1.0007x over previous
import functools

import jax
import jax.numpy as jnp
from jax import lax
from jax.experimental import pallas as pl
from jax.experimental.pallas import tpu as pltpu
from jax.experimental.pallas import tpu_sc as plsc

VOCAB = 100000
EMBED_DIM = 128
BATCH = 4096
HIST = 50

NUM_CORES = 2
NUM_SUBCORES = 16
NW = NUM_CORES * NUM_SUBCORES
BRW = BATCH // NW
NCHUNK = BRW
NBUF = 8
NGRP = NCHUNK // NBUF
LAG = 4

_mesh = plsc.VectorSubcoreMesh(core_axis_name="c", subcore_axis_name="s")


@functools.partial(
    pl.kernel,
    mesh=_mesh,
    out_type=jax.ShapeDtypeStruct((BATCH, HIST, EMBED_DIM), jnp.float32),
    compiler_params=pltpu.CompilerParams(use_tc_tiling_on_sc=True),
    scratch_types=[
        pltpu.VMEM((NCHUNK, HIST), jnp.int32),
    ]
    + [pltpu.VMEM((HIST, EMBED_DIM), jnp.float32) for _ in range(NBUF)]
    + [pltpu.SemaphoreType.DMA for _ in range(2 * NBUF)],
)
def _embedding_lookup(idx_hbm, table_hbm, out_hbm, idx_v, *bufs_and_sems):
    rows = bufs_and_sems[:NBUF]
    gsem = bufs_and_sems[NBUF:2 * NBUF]
    osem = bufs_and_sems[2 * NBUF:]
    wid = lax.axis_index("s") * NUM_CORES + lax.axis_index("c")
    base = wid * BRW
    pltpu.sync_copy(idx_hbm.at[pl.ds(base, BRW)], idx_v)

    def gather(c, j):
        return pltpu.make_async_copy(table_hbm.at[idx_v.at[c]], rows[j], gsem[j])

    def out_copy(c, j):
        return pltpu.make_async_copy(rows[j], out_hbm.at[base + c], osem[j])

    for c in range(NBUF):
        gather(c, c).start()
        if c >= LAG:
            cd = c - LAG
            gather(cd, cd % NBUF).wait()
            out_copy(cd, cd % NBUF).start()

    def body(t, carry):
        c0 = t * NBUF
        for j in range(NBUF):
            c = c0 + j
            out_copy(c - NBUF, j).wait()
            gather(c, j).start()
            cd = c - LAG
            jd = (j - LAG) % NBUF
            gather(cd, jd).wait()
            out_copy(cd, jd).start()
        return carry

    lax.fori_loop(1, NGRP, body, 0)

    for k in range(LAG):
        cd = NCHUNK - LAG + k
        jd = cd % NBUF
        gather(cd, jd).wait()
        out_copy(cd, jd).start()
    for j in range(NBUF):
        out_copy(NCHUNK - NBUF + j, j).wait()


def kernel(inputs, table):
    return _embedding_lookup(inputs.astype(jnp.int32), table)


# speedup vs baseline: 10.4895x; 1.7608x over previous
import functools

import jax
import jax.numpy as jnp
from jax import lax
from jax.experimental import pallas as pl
from jax.experimental.pallas import tpu as pltpu
from jax.experimental.pallas import tpu_sc as plsc

VOCAB = 100000
EMBED_DIM = 128
BATCH = 4096
HIST = 50

NUM_CORES = 2
NUM_SUBCORES = 16
NW = NUM_CORES * NUM_SUBCORES
TOTAL = BATCH * HIST
BPW = TOTAL // NW
CHUNK = 128
NCHUNK = BPW // CHUNK
NBUF = 5
NGRP = NCHUNK // NBUF
LAG = 3

_mesh = plsc.VectorSubcoreMesh(core_axis_name="c", subcore_axis_name="s")


@functools.partial(
    pl.kernel,
    mesh=_mesh,
    out_type=jax.ShapeDtypeStruct((TOTAL, EMBED_DIM), jnp.float32),
    scratch_types=[
        pltpu.VMEM((NCHUNK + 6, CHUNK), jnp.int32),
    ]
    + [pltpu.VMEM((CHUNK, EMBED_DIM), jnp.float32) for _ in range(NBUF)]
    + [pltpu.SemaphoreType.DMA for _ in range(2 * NBUF)],
)
def _embedding_lookup(idx_hbm, table_hbm, out_hbm, idx_v, *bufs_and_sems):
    rows = bufs_and_sems[:NBUF]
    gsem = bufs_and_sems[NBUF:2 * NBUF]
    osem = bufs_and_sems[2 * NBUF:]
    wid = lax.axis_index("s") * NUM_CORES + lax.axis_index("c")
    base = wid * BPW
    roff = lax.rem(wid * NCHUNK, 8)
    start = pl.multiple_of(wid * NCHUNK - roff, 8)
    pltpu.sync_copy(idx_hbm.at[pl.ds(start, NCHUNK + 6)], idx_v)

    def gather(c, j):
        return pltpu.make_async_copy(
            table_hbm.at[idx_v.at[roff + c]], rows[j], gsem[j])

    def out_copy(c, j):
        return pltpu.make_async_copy(
            rows[j], out_hbm.at[pl.ds(base + c * CHUNK, CHUNK)], osem[j])

    for c in range(NBUF):
        gather(c, c).start()
        if c >= LAG:
            cd = c - LAG
            gather(cd, cd % NBUF).wait()
            out_copy(cd, cd % NBUF).start()

    def body(t, carry):
        c0 = t * NBUF
        for j in range(NBUF):
            c = c0 + j
            out_copy(c - NBUF, j).wait()
            gather(c, j).start()
            cd = c - LAG
            jd = (j - LAG) % NBUF
            gather(cd, jd).wait()
            out_copy(cd, jd).start()
        return carry

    lax.fori_loop(1, NGRP, body, 0)

    for k in range(LAG):
        cd = NCHUNK - LAG + k
        jd = cd % NBUF
        gather(cd, jd).wait()
        out_copy(cd, jd).start()
    for j in range(NBUF):
        out_copy(NCHUNK - NBUF + j, j).wait()


def kernel(inputs, table):
    idx = inputs.astype(jnp.int32).T.reshape(TOTAL // CHUNK, CHUNK)
    out = _embedding_lookup(idx, table)
    return out.reshape(HIST, BATCH, EMBED_DIM).transpose(1, 0, 2)
